# baseline (speedup 1.0000x reference)
import functools

import jax
import jax.numpy as jnp
from jax import lax
from jax.experimental import pallas as pl
from jax.experimental.pallas import tpu as pltpu
from jax.experimental.pallas import tpu_sc as plsc

_N = 10000
_E = 160000
_D = 256
_NW = 32
_C = 64
_CPW = 84
_NCID = _NW * _CPW
_LAST = _N + _E - _C


@functools.partial(
    pl.kernel,
    mesh=plsc.VectorSubcoreMesh(core_axis_name="c", subcore_axis_name="s"),
    out_type=jax.ShapeDtypeStruct((_N + _E, _D), jnp.float32),
    scratch_types=[
        pltpu.VMEM((2 * _C,), jnp.int32),
        pltpu.VMEM((2 * _C,), jnp.int32),
        pltpu.VMEM((_C, _D), jnp.float32),
        pltpu.VMEM((_C, _D), jnp.float32),
        pltpu.VMEM((_C, _D), jnp.float32),
        pltpu.VMEM((_C, _D), jnp.float32),
        pltpu.VMEM((_C, _D), jnp.float32),
        pltpu.VMEM((_C, _D), jnp.float32),
        pltpu.SemaphoreType.DMA,
        pltpu.SemaphoreType.DMA,
        pltpu.SemaphoreType.DMA,
        pltpu.SemaphoreType.DMA,
        pltpu.SemaphoreType.DMA,
        pltpu.SemaphoreType.DMA,
    ],
)
def _unpool_kernel(table, idx0, idx1, out, idxa, idxb,
                   rows0a, rows1a, rows0b, rows1b, stga, stgb,
                   isema, isemb, gsema, gsemb, ssema, ssemb):
    w = lax.axis_index("s") * 2 + lax.axis_index("c")

    sets = ((idxa, rows0a, rows1a, stga, isema, gsema, ssema),
            (idxb, rows0b, rows1b, stgb, isemb, gsemb, ssemb))

    def idxload(b, g):
        idxv, _, _, _, isem, _, _ = sets[b]
        cid = w + g * _NW
        base = jnp.minimum(cid * _C, _LAST)
        c0 = pltpu.make_async_copy(
            idx0.at[pl.ds(base, _C)], idxv.at[pl.ds(0, _C)], isem)
        c1 = pltpu.make_async_copy(
            idx1.at[pl.ds(base, _C)], idxv.at[pl.ds(_C, _C)], isem)
        return c0, c1

    def gathers(b):
        idxv, rows0, rows1, _, _, gsem, _ = sets[b]
        c0 = pltpu.make_async_copy(
            table.at[idxv.at[pl.ds(0, _C)]], rows0, gsem)
        c1 = pltpu.make_async_copy(
            table.at[idxv.at[pl.ds(_C, _C)]], rows1, gsem)
        return c0, c1

    def store(b, g):
        _, _, _, stg, _, _, ssem = sets[b]
        cid = w + g * _NW
        base = jnp.minimum(cid * _C, _LAST)
        return pltpu.make_async_copy(stg, out.at[pl.ds(base, _C)], ssem)

    for b in range(2):
        i0, i1 = idxload(b, b)
        i0.start()
        i1.start()
    for b in range(2):
        i0, i1 = idxload(b, b)
        i0.wait()
        i1.wait()
        c0, c1 = gathers(b)
        c0.start()
        c1.start()

    def chunk_step(g, carry):
        for b in range(2):
            @pl.when(g % 2 == b)
            def _():
                _, rows0, rows1, stg, _, _, _ = sets[b]
                c0, c1 = gathers(b)
                c0.wait()
                c1.wait()

                @pl.when(g + 2 < _CPW)
                def _():
                    i0, i1 = idxload(b, g + 2)
                    i0.start()
                    i1.start()

                @pl.when(g >= 2)
                def _():
                    store(b, g - 2).wait()

                def row(r, c2):
                    for j in range(_D // 16):
                        sl = pl.ds(j * 16, 16)
                        stg[r, sl] = (rows0[r, sl] + rows1[r, sl]) * 0.5
                    return c2

                lax.fori_loop(0, _C, row, 0)
                store(b, g).start()

                @pl.when(g + 2 < _CPW)
                def _():
                    i0, i1 = idxload(b, g + 2)
                    i0.wait()
                    i1.wait()
                    n0, n1 = gathers(b)
                    n0.start()
                    n1.start()
        return carry

    lax.fori_loop(0, _CPW, chunk_step, 0)

    store(0, 0).wait()
    store(1, 0).wait()


def kernel(inputs, unpool_idx):
    table = inputs.reshape(_N, _D)
    idx = unpool_idx.astype(jnp.int32)
    self_ids = jnp.arange(_N, dtype=jnp.int32)
    idx0 = jnp.concatenate([self_ids, idx[:, 0]])
    idx1 = jnp.concatenate([self_ids, idx[:, 1]])
    out = _unpool_kernel(table, idx0, idx1)
    return out[None]


# speedup vs baseline: 1.0045x; 1.0045x over previous
import functools

import jax
import jax.numpy as jnp
from jax import lax
from jax.experimental import pallas as pl
from jax.experimental.pallas import tpu as pltpu
from jax.experimental.pallas import tpu_sc as plsc

_N = 10000
_E = 160000
_D = 256
_NW = 32
_C = 120
_CPW = 45
_LAST = _N + _E - _C


@functools.partial(
    pl.kernel,
    mesh=plsc.VectorSubcoreMesh(core_axis_name="c", subcore_axis_name="s"),
    out_type=jax.ShapeDtypeStruct((_N + _E, _D), jnp.float32),
    scratch_types=[
        pltpu.VMEM((2 * _C,), jnp.int32),
        pltpu.VMEM((2 * _C,), jnp.int32),
        pltpu.VMEM((_C, _D), jnp.float32),
        pltpu.VMEM((_C, _D), jnp.float32),
        pltpu.VMEM((_C, _D), jnp.float32),
        pltpu.VMEM((_C, _D), jnp.float32),
        pltpu.SemaphoreType.DMA,
        pltpu.SemaphoreType.DMA,
        pltpu.SemaphoreType.DMA,
        pltpu.SemaphoreType.DMA,
        pltpu.SemaphoreType.DMA,
        pltpu.SemaphoreType.DMA,
    ],
)
def _unpool_kernel(table, idx0, idx1, out, idxa, idxb,
                   rows0a, rows1a, rows0b, rows1b,
                   isema, isemb, gsema, gsemb, ssema, ssemb):
    w = lax.axis_index("s") * 2 + lax.axis_index("c")

    sets = ((idxa, rows0a, rows1a, isema, gsema, ssema),
            (idxb, rows0b, rows1b, isemb, gsemb, ssemb))

    def idxload(b, g):
        idxv, _, _, isem, _, _ = sets[b]
        cid = w + g * _NW
        base = jnp.minimum(cid * _C, _LAST)
        c0 = pltpu.make_async_copy(
            idx0.at[pl.ds(base, _C)], idxv.at[pl.ds(0, _C)], isem)
        c1 = pltpu.make_async_copy(
            idx1.at[pl.ds(base, _C)], idxv.at[pl.ds(_C, _C)], isem)
        return c0, c1

    def gathers(b):
        idxv, rows0, rows1, _, gsem, _ = sets[b]
        c0 = pltpu.make_async_copy(
            table.at[idxv.at[pl.ds(0, _C)]], rows0, gsem)
        c1 = pltpu.make_async_copy(
            table.at[idxv.at[pl.ds(_C, _C)]], rows1, gsem)
        return c0, c1

    def store(b, g):
        _, rows0, _, _, _, ssem = sets[b]
        cid = w + g * _NW
        base = jnp.minimum(cid * _C, _LAST)
        return pltpu.make_async_copy(rows0, out.at[pl.ds(base, _C)], ssem)

    for b in range(2):
        i0, i1 = idxload(b, b)
        i0.start()
        i1.start()
    for b in range(2):
        i0, i1 = idxload(b, b)
        i0.wait()
        i1.wait()
        c0, c1 = gathers(b)
        c0.start()
        c1.start()

    def chunk_step(g, carry):
        for b in range(2):
            @pl.when(g % 2 == b)
            def _():
                _, rows0, rows1, _, _, _ = sets[b]
                c0, c1 = gathers(b)
                c0.wait()
                c1.wait()

                @pl.when(g + 2 < _CPW)
                def _():
                    i0, i1 = idxload(b, g + 2)
                    i0.start()
                    i1.start()

                def row(r, c2):
                    for j in range(_D // 16):
                        sl = pl.ds(j * 16, 16)
                        rows0[r, sl] = (rows0[r, sl] + rows1[r, sl]) * 0.5
                    return c2

                lax.fori_loop(0, _C, row, 0)
                store(b, g).start()

                @pl.when(g + 2 < _CPW)
                def _():
                    store(b, g).wait()
                    i0, i1 = idxload(b, g + 2)
                    i0.wait()
                    i1.wait()
                    n0, n1 = gathers(b)
                    n0.start()
                    n1.start()
        return carry

    lax.fori_loop(0, _CPW, chunk_step, 0)

    store(0, 0).wait()
    store(1, 0).wait()


def kernel(inputs, unpool_idx):
    table = inputs.reshape(_N, _D)
    idx = unpool_idx.astype(jnp.int32)
    self_ids = jnp.arange(_N, dtype=jnp.int32)
    idx0 = jnp.concatenate([self_ids, idx[:, 0]])
    idx1 = jnp.concatenate([self_ids, idx[:, 1]])
    out = _unpool_kernel(table, idx0, idx1)
    return out[None]
